# baseline (speedup 1.0000x reference)
import functools

import jax
import jax.numpy as jnp
from jax import lax
from jax.experimental import pallas as pl
from jax.experimental.pallas import tpu as pltpu
from jax.experimental.pallas import tpu_sc as plsc

F = 512
NB = 16
BOUND = 5.0
MIN_DERIV = 0.001
MIN_BW = 0.001
MIN_BH = 0.001
BATCH = 16384

TOT = BATCH * F
NW = 32
PER_W = TOT // NW
CHUNK = 8192
NCHUNK = PER_W // CHUNK
NVEC = CHUNK // 16
CSTRIDE = F * 18


def _softmax_ax0(a):
    m = jnp.max(a, axis=0, keepdims=True)
    e = jnp.exp(a - m)
    return e / jnp.sum(e, axis=0, keepdims=True)


def _prep_body(uw_ref, uh_ref, ud_ref, bounds_ref, coef_ref):
    uw = uw_ref[...]
    uh = uh_ref[...]
    ud = ud_ref[...]

    widths = _softmax_ax0(uw)
    widths = MIN_BW + (1.0 - MIN_BW * NB) * widths
    widths = widths * (2.0 * BOUND)
    heights = _softmax_ax0(uh)
    heights = MIN_BH + (1.0 - MIN_BH * NB) * heights
    heights = heights * (2.0 * BOUND)
    deriv = jnp.maximum(ud, 0.0) + jnp.log(1.0 + jnp.exp(-jnp.abs(ud))) + MIN_DERIV

    r = lax.broadcasted_iota(jnp.int32, (NB, NB), 0)
    c = lax.broadcasted_iota(jnp.int32, (NB, NB), 1)
    tri = (c <= r).astype(jnp.float32)
    cum_w = jnp.dot(tri, widths, precision=lax.Precision.HIGHEST,
                    preferred_element_type=jnp.float32)
    cum_h = jnp.dot(tri, heights, precision=lax.Precision.HIGHEST,
                    preferred_element_type=jnp.float32)

    neg_b = jnp.full((1, F), -BOUND, jnp.float32)
    knot_x = jnp.concatenate([neg_b, -BOUND + cum_w], axis=0)
    knot_y = jnp.concatenate([neg_b, -BOUND + cum_h], axis=0)

    bounds_ref[...] = jnp.concatenate(
        [knot_x[1:16], jnp.full((1, F), 1e30, jnp.float32)], axis=0)

    s = heights / widths
    d0 = deriv[:NB]
    d1 = deriv[1:]
    p = heights * (s - d0) / s
    q = heights * d0 / s
    c2 = (d0 + d1 - 2.0 * s) / s

    one = jnp.ones((1, F), jnp.float32)
    zero = jnp.zeros((1, F), jnp.float32)

    def plane(row0, inner, row17):
        return jnp.concatenate([row0, inner, row17], axis=0)

    coef = jnp.stack([
        plane(knot_x[0:1], knot_x[:NB], knot_x[16:17]),
        plane(one, 1.0 / widths, one),
        plane(zero, p, zero),
        plane(deriv[0:1], q, deriv[16:17]),
        plane(zero, c2, zero),
        plane(knot_y[0:1], knot_y[:NB], knot_y[16:17]),
    ], axis=0)
    coef_ref[...] = coef


_prep = pl.pallas_call(
    _prep_body,
    out_shape=[
        jax.ShapeDtypeStruct((16, F), jnp.float32),
        jax.ShapeDtypeStruct((6, 18, F), jnp.float32),
    ],
)


def _sc_body(x_hbm, bounds_hbm, coef_hbm, y_hbm,
             bnd_v, cf0, cf1, cf2, cf3, cf4, cf5,
             xb0, xb1, yb0, yb1, si0, si1, so0, so1):
    wid = lax.axis_index("s") * 2 + lax.axis_index("c")
    base = pl.multiple_of(wid * PER_W, CHUNK)

    pltpu.sync_copy(bounds_hbm, bnd_v)
    cfs = (cf0, cf1, cf2, cf3, cf4, cf5)
    for k in range(6):
        pltpu.sync_copy(coef_hbm.at[pl.ds(k * CSTRIDE, CSTRIDE)], cfs[k])

    xbufs = (xb0, xb1)
    ybufs = (yb0, yb1)
    isems = (si0, si1)
    osems = (so0, so1)

    def in_slice(g):
        return x_hbm.at[pl.ds(pl.multiple_of(base + g * CHUNK, CHUNK), CHUNK)]

    def out_slice(g):
        return y_hbm.at[pl.ds(pl.multiple_of(base + g * CHUNK, CHUNK), CHUNK)]

    io16 = lax.broadcasted_iota(jnp.int32, (16,), 0)

    def compute(b):
        xbuf = xbufs[b]
        ybuf = ybufs[b]

        @plsc.parallel_loop(0, NVEC, unroll=8)
        def body(i):
            off = i * 16
            fb = jnp.bitwise_and(off, F - 1)
            f = fb + io16
            xv = xbuf[pl.ds(off, 16)]
            acc = f
            for stp in (8, 4, 2, 1):
                probe = plsc.load_gather(bnd_v, [acc + (stp - 1) * F])
                acc = jnp.where(xv >= probe, acc + stp * F, acc)
            cb = acc + jnp.where(xv < -BOUND, 0, F)
            cb = cb + jnp.where(xv > BOUND, F, 0)
            xk = plsc.load_gather(cf0, [cb])
            iw = plsc.load_gather(cf1, [cb])
            pp = plsc.load_gather(cf2, [cb])
            qq = plsc.load_gather(cf3, [cb])
            cc = plsc.load_gather(cf4, [cb])
            yk = plsc.load_gather(cf5, [cb])
            xi = (xv - xk) * iw
            t = xi * (1.0 - xi)
            num = xi * (pp * xi + qq)
            den = 1.0 + cc * t
            ybuf[pl.ds(off, 16)] = yk + num / den

    pltpu.async_copy(in_slice(0), xb0, si0)
    pltpu.async_copy(in_slice(1), xb1, si1)

    def outer(k, carry):
        g2 = k * 2
        for b in (0, 1):
            g = g2 + b
            pltpu.make_async_copy(in_slice(0), xbufs[b], isems[b]).wait()

            @pl.when(g2 >= 2)
            def _wait_out():
                pltpu.make_async_copy(ybufs[b], out_slice(0), osems[b]).wait()

            compute(b)
            pltpu.async_copy(ybufs[b], out_slice(g), osems[b])

            @pl.when(g2 + 2 < NCHUNK)
            def _next_in():
                pltpu.async_copy(in_slice(g + 2), xbufs[b], isems[b])
        return carry

    lax.fori_loop(0, NCHUNK // 2, outer, 0)

    pltpu.make_async_copy(yb0, out_slice(0), so0).wait()
    pltpu.make_async_copy(yb1, out_slice(0), so1).wait()


_sc_eval = functools.partial(
    pl.kernel,
    out_type=jax.ShapeDtypeStruct((TOT,), jnp.float32),
    mesh=plsc.VectorSubcoreMesh(core_axis_name="c", subcore_axis_name="s"),
    scratch_types=[
        pltpu.VMEM((F * 16,), jnp.float32),
        pltpu.VMEM((CSTRIDE,), jnp.float32),
        pltpu.VMEM((CSTRIDE,), jnp.float32),
        pltpu.VMEM((CSTRIDE,), jnp.float32),
        pltpu.VMEM((CSTRIDE,), jnp.float32),
        pltpu.VMEM((CSTRIDE,), jnp.float32),
        pltpu.VMEM((CSTRIDE,), jnp.float32),
        pltpu.VMEM((CHUNK,), jnp.float32),
        pltpu.VMEM((CHUNK,), jnp.float32),
        pltpu.VMEM((CHUNK,), jnp.float32),
        pltpu.VMEM((CHUNK,), jnp.float32),
        pltpu.SemaphoreType.DMA,
        pltpu.SemaphoreType.DMA,
        pltpu.SemaphoreType.DMA,
        pltpu.SemaphoreType.DMA,
    ],
    compiler_params=pltpu.CompilerParams(needs_layout_passes=False),
)(_sc_body)


def kernel(x, unnorm_widths, unnorm_heights, unnorm_derivatives):
    bounds, coef = _prep(unnorm_widths.T, unnorm_heights.T,
                         unnorm_derivatives.T)
    y = _sc_eval(x.reshape(-1), bounds.reshape(-1), coef.reshape(-1))
    return y.reshape(BATCH, F)


# speedup vs baseline: 1.0360x; 1.0360x over previous
import functools

import jax
import jax.numpy as jnp
from jax import lax
from jax.experimental import pallas as pl
from jax.experimental.pallas import tpu as pltpu
from jax.experimental.pallas import tpu_sc as plsc

F = 512
NB = 16
BOUND = 5.0
MIN_DERIV = 0.001
MIN_BW = 0.001
MIN_BH = 0.001
BATCH = 16384

B1 = 7168
B2 = BATCH - B1
TBLK = 1024

TOT = B2 * F
NW = 32
PER_W = TOT // NW
CHUNK = 8192
NCHUNK = PER_W // CHUNK
NVEC = CHUNK // 16
CSTRIDE = F * 18
assert PER_W % CHUNK == 0 and NCHUNK % 2 == 0 and B1 % TBLK == 0


def _softmax_ax0(a):
    m = jnp.max(a, axis=0, keepdims=True)
    e = jnp.exp(a - m)
    return e / jnp.sum(e, axis=0, keepdims=True)


def _prep_body(uw_ref, uh_ref, ud_ref, bounds_ref, coef_ref):
    uw = uw_ref[...]
    uh = uh_ref[...]
    ud = ud_ref[...]

    widths = _softmax_ax0(uw)
    widths = MIN_BW + (1.0 - MIN_BW * NB) * widths
    widths = widths * (2.0 * BOUND)
    heights = _softmax_ax0(uh)
    heights = MIN_BH + (1.0 - MIN_BH * NB) * heights
    heights = heights * (2.0 * BOUND)
    deriv = jnp.maximum(ud, 0.0) + jnp.log(1.0 + jnp.exp(-jnp.abs(ud))) + MIN_DERIV

    r = lax.broadcasted_iota(jnp.int32, (NB, NB), 0)
    c = lax.broadcasted_iota(jnp.int32, (NB, NB), 1)
    tri = (c <= r).astype(jnp.float32)
    cum_w = jnp.dot(tri, widths, precision=lax.Precision.HIGHEST,
                    preferred_element_type=jnp.float32)
    cum_h = jnp.dot(tri, heights, precision=lax.Precision.HIGHEST,
                    preferred_element_type=jnp.float32)

    neg_b = jnp.full((1, F), -BOUND, jnp.float32)
    knot_x = jnp.concatenate([neg_b, -BOUND + cum_w], axis=0)
    knot_y = jnp.concatenate([neg_b, -BOUND + cum_h], axis=0)

    bounds_ref[...] = jnp.concatenate(
        [knot_x[1:16], jnp.full((1, F), 1e30, jnp.float32)], axis=0)

    s = heights / widths
    d0 = deriv[:NB]
    d1 = deriv[1:]
    p = heights * (s - d0) / s
    q = heights * d0 / s
    c2 = (d0 + d1 - 2.0 * s) / s

    one = jnp.ones((1, F), jnp.float32)
    zero = jnp.zeros((1, F), jnp.float32)

    def plane(row0, inner, row17):
        return jnp.concatenate([row0, inner, row17], axis=0)

    coef = jnp.stack([
        plane(knot_x[0:1], knot_x[:NB], knot_x[16:17]),
        plane(one, 1.0 / widths, one),
        plane(zero, p, zero),
        plane(deriv[0:1], q, deriv[16:17]),
        plane(zero, c2, zero),
        plane(knot_y[0:1], knot_y[:NB], knot_y[16:17]),
    ], axis=0)
    coef_ref[...] = coef


_prep = pl.pallas_call(
    _prep_body,
    out_shape=[
        jax.ShapeDtypeStruct((16, F), jnp.float32),
        jax.ShapeDtypeStruct((6, 18, F), jnp.float32),
    ],
)


def _sc_body(x_hbm, bounds_hbm, coef_hbm, y_hbm,
             bnd_v, cf0, cf1, cf2, cf3, cf4, cf5,
             xb0, xb1, yb0, yb1, si0, si1, so0, so1):
    wid = lax.axis_index("s") * 2 + lax.axis_index("c")
    base = pl.multiple_of(wid * PER_W, CHUNK)

    pltpu.sync_copy(bounds_hbm, bnd_v)
    cfs = (cf0, cf1, cf2, cf3, cf4, cf5)
    for k in range(6):
        pltpu.sync_copy(coef_hbm.at[pl.ds(k * CSTRIDE, CSTRIDE)], cfs[k])

    xbufs = (xb0, xb1)
    ybufs = (yb0, yb1)
    isems = (si0, si1)
    osems = (so0, so1)

    def in_slice(g):
        return x_hbm.at[pl.ds(pl.multiple_of(base + g * CHUNK, CHUNK), CHUNK)]

    def out_slice(g):
        return y_hbm.at[pl.ds(pl.multiple_of(base + g * CHUNK, CHUNK), CHUNK)]

    io16 = lax.broadcasted_iota(jnp.int32, (16,), 0)

    def compute(b):
        xbuf = xbufs[b]
        ybuf = ybufs[b]

        @plsc.parallel_loop(0, NVEC, unroll=4)
        def body(i):
            off = i * 16
            fb = jnp.bitwise_and(off, F - 1)
            f = fb + io16
            xv = xbuf[pl.ds(off, 16)]
            acc = f
            for stp in (8, 4, 2, 1):
                probe = plsc.load_gather(bnd_v, [acc + (stp - 1) * F])
                acc = jnp.where(xv >= probe, acc + stp * F, acc)
            cb = acc + jnp.where(xv < -BOUND, 0, F)
            cb = cb + jnp.where(xv > BOUND, F, 0)
            xk = plsc.load_gather(cf0, [cb])
            iw = plsc.load_gather(cf1, [cb])
            pp = plsc.load_gather(cf2, [cb])
            qq = plsc.load_gather(cf3, [cb])
            cc = plsc.load_gather(cf4, [cb])
            yk = plsc.load_gather(cf5, [cb])
            xi = (xv - xk) * iw
            t = xi * (1.0 - xi)
            num = xi * (pp * xi + qq)
            den = 1.0 + cc * t
            ybuf[pl.ds(off, 16)] = yk + num / den

    pltpu.async_copy(in_slice(0), xb0, si0)
    pltpu.async_copy(in_slice(1), xb1, si1)

    def outer(k, carry):
        g2 = k * 2
        for b in (0, 1):
            g = g2 + b
            pltpu.make_async_copy(in_slice(0), xbufs[b], isems[b]).wait()

            @pl.when(g2 >= 2)
            def _wait_out():
                pltpu.make_async_copy(ybufs[b], out_slice(0), osems[b]).wait()

            compute(b)
            pltpu.async_copy(ybufs[b], out_slice(g), osems[b])

            @pl.when(g2 + 2 < NCHUNK)
            def _next_in():
                pltpu.async_copy(in_slice(g + 2), xbufs[b], isems[b])
        return carry

    lax.fori_loop(0, NCHUNK // 2, outer, 0)

    pltpu.make_async_copy(yb0, out_slice(0), so0).wait()
    pltpu.make_async_copy(yb1, out_slice(0), so1).wait()


_sc_eval = functools.partial(
    pl.kernel,
    out_type=jax.ShapeDtypeStruct((TOT,), jnp.float32),
    mesh=plsc.VectorSubcoreMesh(core_axis_name="c", subcore_axis_name="s"),
    scratch_types=[
        pltpu.VMEM((F * 16,), jnp.float32),
        pltpu.VMEM((CSTRIDE,), jnp.float32),
        pltpu.VMEM((CSTRIDE,), jnp.float32),
        pltpu.VMEM((CSTRIDE,), jnp.float32),
        pltpu.VMEM((CSTRIDE,), jnp.float32),
        pltpu.VMEM((CSTRIDE,), jnp.float32),
        pltpu.VMEM((CSTRIDE,), jnp.float32),
        pltpu.VMEM((CHUNK,), jnp.float32),
        pltpu.VMEM((CHUNK,), jnp.float32),
        pltpu.VMEM((CHUNK,), jnp.float32),
        pltpu.VMEM((CHUNK,), jnp.float32),
        pltpu.SemaphoreType.DMA,
        pltpu.SemaphoreType.DMA,
        pltpu.SemaphoreType.DMA,
        pltpu.SemaphoreType.DMA,
    ],
    compiler_params=pltpu.CompilerParams(needs_layout_passes=False),
)(_sc_body)


def _tc_eval_body(x_ref, bnd_ref, coef_ref, y_ref):
    x = x_ref[...]
    cf = coef_ref[...]
    accs = [jnp.broadcast_to(cf[k, 0:1, :], (TBLK, F)) for k in range(6)]
    for j in range(17):
        if j == 0:
            m = x >= -BOUND
        elif j <= 15:
            m = x >= bnd_ref[j - 1:j, :]
        else:
            m = x > BOUND
        for k in range(6):
            d = cf[k, j + 1:j + 2, :] - cf[k, j:j + 1, :]
            accs[k] = accs[k] + jnp.where(m, d, 0.0)
    xk, iw, pp, qq, cc, yk = accs
    xi = (x - xk) * iw
    t = xi * (1.0 - xi)
    num = xi * (pp * xi + qq)
    den = 1.0 + cc * t
    y_ref[...] = yk + num / den


_tc_eval = pl.pallas_call(
    _tc_eval_body,
    grid=(B1 // TBLK,),
    in_specs=[
        pl.BlockSpec((TBLK, F), lambda i: (i, 0)),
        pl.BlockSpec((16, F), lambda i: (0, 0)),
        pl.BlockSpec((6, 18, F), lambda i: (0, 0, 0)),
    ],
    out_specs=pl.BlockSpec((TBLK, F), lambda i: (i, 0)),
    out_shape=jax.ShapeDtypeStruct((B1, F), jnp.float32),
)


def kernel(x, unnorm_widths, unnorm_heights, unnorm_derivatives):
    bounds, coef = _prep(unnorm_widths.T, unnorm_heights.T,
                         unnorm_derivatives.T)
    y2 = _sc_eval(x[B1:].reshape(-1), bounds.reshape(-1), coef.reshape(-1))
    y1 = _tc_eval(x[:B1], bounds, coef)
    return jnp.concatenate([y1, y2.reshape(B2, F)], axis=0)


# speedup vs baseline: 1.0369x; 1.0008x over previous
import functools

import jax
import jax.numpy as jnp
from jax import lax
from jax.experimental import pallas as pl
from jax.experimental.pallas import tpu as pltpu
from jax.experimental.pallas import tpu_sc as plsc

F = 512
NB = 16
BOUND = 5.0
MIN_DERIV = 0.001
MIN_BW = 0.001
MIN_BH = 0.001
BATCH = 16384

B1 = 7168
B2 = BATCH - B1
TBLK = 1024

TOT = B2 * F
NW = 32
PER_W = TOT // NW
CHUNK = 8192
NCHUNK = PER_W // CHUNK
NVEC = CHUNK // 16
CSTRIDE = F * 18
assert PER_W % CHUNK == 0 and NCHUNK % 2 == 0 and B1 % TBLK == 0


def _softmax_ax0(a):
    m = jnp.max(a, axis=0, keepdims=True)
    e = jnp.exp(a - m)
    return e / jnp.sum(e, axis=0, keepdims=True)


def _prep_body(uw_ref, uh_ref, ud_ref, bounds_ref, coef_ref):
    uw = uw_ref[...]
    uh = uh_ref[...]
    ud = ud_ref[...]

    widths = _softmax_ax0(uw)
    widths = MIN_BW + (1.0 - MIN_BW * NB) * widths
    widths = widths * (2.0 * BOUND)
    heights = _softmax_ax0(uh)
    heights = MIN_BH + (1.0 - MIN_BH * NB) * heights
    heights = heights * (2.0 * BOUND)
    deriv = jnp.maximum(ud, 0.0) + jnp.log(1.0 + jnp.exp(-jnp.abs(ud))) + MIN_DERIV

    r = lax.broadcasted_iota(jnp.int32, (NB, NB), 0)
    c = lax.broadcasted_iota(jnp.int32, (NB, NB), 1)
    tri = (c <= r).astype(jnp.float32)
    cum_w = jnp.dot(tri, widths, precision=lax.Precision.HIGHEST,
                    preferred_element_type=jnp.float32)
    cum_h = jnp.dot(tri, heights, precision=lax.Precision.HIGHEST,
                    preferred_element_type=jnp.float32)

    neg_b = jnp.full((1, F), -BOUND, jnp.float32)
    knot_x = jnp.concatenate([neg_b, -BOUND + cum_w], axis=0)
    knot_y = jnp.concatenate([neg_b, -BOUND + cum_h], axis=0)

    bounds_ref[...] = jnp.concatenate(
        [knot_x[1:16], jnp.full((1, F), 1e30, jnp.float32)], axis=0)

    s = heights / widths
    d0 = deriv[:NB]
    d1 = deriv[1:]
    p = heights * (s - d0) / s
    q = heights * d0 / s
    c2 = (d0 + d1 - 2.0 * s) / s

    one = jnp.ones((1, F), jnp.float32)
    zero = jnp.zeros((1, F), jnp.float32)

    def plane(row0, inner, row17):
        return jnp.concatenate([row0, inner, row17], axis=0)

    coef = jnp.stack([
        plane(knot_x[0:1], knot_x[:NB], knot_x[16:17]),
        plane(one, 1.0 / widths, one),
        plane(zero, p, zero),
        plane(deriv[0:1], q, deriv[16:17]),
        plane(zero, c2, zero),
        plane(knot_y[0:1], knot_y[:NB], knot_y[16:17]),
    ], axis=0)
    coef_ref[...] = coef


_prep = pl.pallas_call(
    _prep_body,
    out_shape=[
        jax.ShapeDtypeStruct((16, F), jnp.float32),
        jax.ShapeDtypeStruct((6, 18, F), jnp.float32),
    ],
)


def _sc_body(x_hbm, bounds_hbm, coef_hbm, y_hbm,
             bnd_v, cf0, cf1, cf2, cf3, cf4, cf5,
             xb0, xb1, yb0, yb1, si0, si1, so0, so1):
    wid = lax.axis_index("s") * 2 + lax.axis_index("c")
    base = pl.multiple_of(wid * PER_W, CHUNK)

    pltpu.sync_copy(bounds_hbm, bnd_v)
    cfs = (cf0, cf1, cf2, cf3, cf4, cf5)
    for k in range(6):
        pltpu.sync_copy(coef_hbm.at[pl.ds(k * CSTRIDE, CSTRIDE)], cfs[k])

    xbufs = (xb0, xb1)
    ybufs = (yb0, yb1)
    isems = (si0, si1)
    osems = (so0, so1)

    def in_slice(g):
        return x_hbm.at[pl.ds(pl.multiple_of(base + g * CHUNK, CHUNK), CHUNK)]

    def out_slice(g):
        return y_hbm.at[pl.ds(pl.multiple_of(base + g * CHUNK, CHUNK), CHUNK)]

    io16 = lax.broadcasted_iota(jnp.int32, (16,), 0)

    def compute(b):
        xbuf = xbufs[b]
        ybuf = ybufs[b]

        @plsc.parallel_loop(0, NVEC, unroll=4)
        def body(i):
            off = i * 16
            fb = jnp.bitwise_and(off, F - 1)
            f = fb + io16
            xv = xbuf[pl.ds(off, 16)]
            acc = f
            for stp in (8, 4, 2, 1):
                probe = plsc.load_gather(bnd_v, [acc + (stp - 1) * F])
                acc = jnp.where(xv >= probe, acc + stp * F, acc)
            cb = acc + jnp.where(xv < -BOUND, 0, F)
            cb = cb + jnp.where(xv > BOUND, F, 0)
            xk = plsc.load_gather(cf0, [cb])
            iw = plsc.load_gather(cf1, [cb])
            pp = plsc.load_gather(cf2, [cb])
            qq = plsc.load_gather(cf3, [cb])
            cc = plsc.load_gather(cf4, [cb])
            yk = plsc.load_gather(cf5, [cb])
            xi = (xv - xk) * iw
            t = xi * (1.0 - xi)
            num = xi * (pp * xi + qq)
            den = 1.0 + cc * t
            ybuf[pl.ds(off, 16)] = yk + num / den

    pltpu.async_copy(in_slice(0), xb0, si0)
    pltpu.async_copy(in_slice(1), xb1, si1)

    def outer(k, carry):
        g2 = k * 2
        for b in (0, 1):
            g = g2 + b
            pltpu.make_async_copy(in_slice(0), xbufs[b], isems[b]).wait()

            @pl.when(g2 >= 2)
            def _wait_out():
                pltpu.make_async_copy(ybufs[b], out_slice(0), osems[b]).wait()

            compute(b)
            pltpu.async_copy(ybufs[b], out_slice(g), osems[b])

            @pl.when(g2 + 2 < NCHUNK)
            def _next_in():
                pltpu.async_copy(in_slice(g + 2), xbufs[b], isems[b])
        return carry

    lax.fori_loop(0, NCHUNK // 2, outer, 0)

    pltpu.make_async_copy(yb0, out_slice(0), so0).wait()
    pltpu.make_async_copy(yb1, out_slice(0), so1).wait()


_sc_eval = functools.partial(
    pl.kernel,
    out_type=jax.ShapeDtypeStruct((TOT,), jnp.float32),
    mesh=plsc.VectorSubcoreMesh(core_axis_name="c", subcore_axis_name="s"),
    scratch_types=[
        pltpu.VMEM((F * 16,), jnp.float32),
        pltpu.VMEM((CSTRIDE,), jnp.float32),
        pltpu.VMEM((CSTRIDE,), jnp.float32),
        pltpu.VMEM((CSTRIDE,), jnp.float32),
        pltpu.VMEM((CSTRIDE,), jnp.float32),
        pltpu.VMEM((CSTRIDE,), jnp.float32),
        pltpu.VMEM((CSTRIDE,), jnp.float32),
        pltpu.VMEM((CHUNK,), jnp.float32),
        pltpu.VMEM((CHUNK,), jnp.float32),
        pltpu.VMEM((CHUNK,), jnp.float32),
        pltpu.VMEM((CHUNK,), jnp.float32),
        pltpu.SemaphoreType.DMA,
        pltpu.SemaphoreType.DMA,
        pltpu.SemaphoreType.DMA,
        pltpu.SemaphoreType.DMA,
    ],
    compiler_params=pltpu.CompilerParams(needs_layout_passes=False),
)(_sc_body)


def _tc_eval_body(x_ref, bnd_ref, coef_ref, y_ref):
    x = x_ref[...]
    cf = coef_ref[...]
    accs = [jnp.broadcast_to(cf[k, 0:1, :], (TBLK, F)) for k in range(6)]
    for j in range(17):
        if j == 0:
            m = x >= -BOUND
        elif j <= 15:
            m = x >= bnd_ref[j - 1:j, :]
        else:
            m = x > BOUND
        for k in range(6):
            d = cf[k, j + 1:j + 2, :] - cf[k, j:j + 1, :]
            accs[k] = accs[k] + jnp.where(m, d, 0.0)
    xk, iw, pp, qq, cc, yk = accs
    xi = (x - xk) * iw
    t = xi * (1.0 - xi)
    num = xi * (pp * xi + qq)
    den = 1.0 + cc * t
    y_ref[...] = yk + num / den


_tc_eval = pl.pallas_call(
    _tc_eval_body,
    grid=(B1 // TBLK,),
    in_specs=[
        pl.BlockSpec((TBLK, F), lambda i: (i, 0)),
        pl.BlockSpec((16, F), lambda i: (0, 0)),
        pl.BlockSpec((6, 18, F), lambda i: (0, 0, 0)),
    ],
    out_specs=pl.BlockSpec((TBLK, F), lambda i: (i, 0)),
    out_shape=jax.ShapeDtypeStruct((B1, F), jnp.float32),
)


def kernel(x, unnorm_widths, unnorm_heights, unnorm_derivatives):
    bounds, coef = _prep(unnorm_widths.T, unnorm_heights.T,
                         unnorm_derivatives.T)
    y1 = _tc_eval(x[:B1], bounds, coef)
    y2 = _sc_eval(x[B1:].reshape(-1), bounds.reshape(-1), coef.reshape(-1))
    return jnp.concatenate([y1, y2.reshape(B2, F)], axis=0)


# speedup vs baseline: 1.1219x; 1.0820x over previous
import functools

import jax
import jax.numpy as jnp
from jax import lax
from jax.experimental import pallas as pl
from jax.experimental.pallas import tpu as pltpu
from jax.experimental.pallas import tpu_sc as plsc

F = 512
NB = 16
BOUND = 5.0
MIN_DERIV = 0.001
MIN_BW = 0.001
MIN_BH = 0.001
BATCH = 16384

TOT = BATCH * F
NW = 32
PER_W = TOT // NW
CHUNK = 16384
NCHUNK = PER_W // CHUNK
NVEC = CHUNK // 16
CSTRIDE = F * 18
assert PER_W % CHUNK == 0 and NCHUNK % 2 == 0


def _softmax_ax0(a):
    m = jnp.max(a, axis=0, keepdims=True)
    e = jnp.exp(a - m)
    return e / jnp.sum(e, axis=0, keepdims=True)


def _prep_body(uw_ref, uh_ref, ud_ref, bounds_ref, coef_ref):
    uw = uw_ref[...]
    uh = uh_ref[...]
    ud = ud_ref[...]

    widths = _softmax_ax0(uw)
    widths = MIN_BW + (1.0 - MIN_BW * NB) * widths
    widths = widths * (2.0 * BOUND)
    heights = _softmax_ax0(uh)
    heights = MIN_BH + (1.0 - MIN_BH * NB) * heights
    heights = heights * (2.0 * BOUND)
    deriv = jnp.maximum(ud, 0.0) + jnp.log(1.0 + jnp.exp(-jnp.abs(ud))) + MIN_DERIV

    r = lax.broadcasted_iota(jnp.int32, (NB, NB), 0)
    c = lax.broadcasted_iota(jnp.int32, (NB, NB), 1)
    tri = (c <= r).astype(jnp.float32)
    cum_w = jnp.dot(tri, widths, precision=lax.Precision.HIGHEST,
                    preferred_element_type=jnp.float32)
    cum_h = jnp.dot(tri, heights, precision=lax.Precision.HIGHEST,
                    preferred_element_type=jnp.float32)

    neg_b = jnp.full((1, F), -BOUND, jnp.float32)
    knot_x = jnp.concatenate([neg_b, -BOUND + cum_w], axis=0)
    knot_y = jnp.concatenate([neg_b, -BOUND + cum_h], axis=0)

    bounds_ref[...] = jnp.concatenate(
        [knot_x[1:16], jnp.full((1, F), 1e30, jnp.float32)], axis=0)

    s = heights / widths
    d0 = deriv[:NB]
    d1 = deriv[1:]
    p = heights * (s - d0) / s
    q = heights * d0 / s
    c2 = (d0 + d1 - 2.0 * s) / s

    one = jnp.ones((1, F), jnp.float32)
    zero = jnp.zeros((1, F), jnp.float32)

    def plane(row0, inner, row17):
        return jnp.concatenate([row0, inner, row17], axis=0)

    coef = jnp.stack([
        plane(knot_x[0:1], knot_x[:NB], knot_x[16:17]),
        plane(one, 1.0 / widths, one),
        plane(zero, p, zero),
        plane(deriv[0:1], q, deriv[16:17]),
        plane(zero, c2, zero),
        plane(knot_y[0:1], knot_y[:NB], knot_y[16:17]),
    ], axis=0)
    coef_ref[...] = coef


_prep = pl.pallas_call(
    _prep_body,
    out_shape=[
        jax.ShapeDtypeStruct((16, F), jnp.float32),
        jax.ShapeDtypeStruct((6, 18, F), jnp.float32),
    ],
)


def _sc_body(x_hbm, bounds_hbm, coef_hbm, y_hbm,
             bnd_v, cf0, cf1, cf2, cf3, cf4, cf5,
             xb0, xb1, yb0, yb1, si0, si1, so0, so1):
    wid = lax.axis_index("s") * 2 + lax.axis_index("c")
    base = pl.multiple_of(wid * PER_W, CHUNK)

    pltpu.sync_copy(bounds_hbm, bnd_v)
    cfs = (cf0, cf1, cf2, cf3, cf4, cf5)
    for k in range(6):
        pltpu.sync_copy(coef_hbm.at[pl.ds(k * CSTRIDE, CSTRIDE)], cfs[k])

    xbufs = (xb0, xb1)
    ybufs = (yb0, yb1)
    isems = (si0, si1)
    osems = (so0, so1)

    def in_slice(g):
        return x_hbm.at[pl.ds(pl.multiple_of(base + g * CHUNK, CHUNK), CHUNK)]

    def out_slice(g):
        return y_hbm.at[pl.ds(pl.multiple_of(base + g * CHUNK, CHUNK), CHUNK)]

    io16 = lax.broadcasted_iota(jnp.int32, (16,), 0)

    def compute(b):
        xbuf = xbufs[b]
        ybuf = ybufs[b]

        @plsc.parallel_loop(0, NVEC, unroll=4)
        def body(i):
            off = i * 16
            fb = jnp.bitwise_and(off, F - 1)
            f = fb + io16
            xv = xbuf[pl.ds(off, 16)]
            acc = f
            for stp in (8, 4, 2, 1):
                probe = plsc.load_gather(bnd_v, [acc + (stp - 1) * F])
                acc = jnp.where(xv >= probe, acc + stp * F, acc)
            cb = acc + jnp.where(xv < -BOUND, 0, F)
            cb = cb + jnp.where(xv > BOUND, F, 0)
            xk = plsc.load_gather(cf0, [cb])
            iw = plsc.load_gather(cf1, [cb])
            pp = plsc.load_gather(cf2, [cb])
            qq = plsc.load_gather(cf3, [cb])
            cc = plsc.load_gather(cf4, [cb])
            yk = plsc.load_gather(cf5, [cb])
            xi = (xv - xk) * iw
            t = xi * (1.0 - xi)
            num = xi * (pp * xi + qq)
            den = 1.0 + cc * t
            ybuf[pl.ds(off, 16)] = yk + num / den

    pltpu.async_copy(in_slice(0), xb0, si0)
    pltpu.async_copy(in_slice(1), xb1, si1)

    def outer(k, carry):
        g2 = k * 2
        for b in (0, 1):
            g = g2 + b
            pltpu.make_async_copy(in_slice(0), xbufs[b], isems[b]).wait()

            @pl.when(g2 >= 2)
            def _wait_out():
                pltpu.make_async_copy(ybufs[b], out_slice(0), osems[b]).wait()

            compute(b)
            pltpu.async_copy(ybufs[b], out_slice(g), osems[b])

            @pl.when(g2 + 2 < NCHUNK)
            def _next_in():
                pltpu.async_copy(in_slice(g + 2), xbufs[b], isems[b])
        return carry

    lax.fori_loop(0, NCHUNK // 2, outer, 0)

    pltpu.make_async_copy(yb0, out_slice(0), so0).wait()
    pltpu.make_async_copy(yb1, out_slice(0), so1).wait()


_sc_eval = functools.partial(
    pl.kernel,
    out_type=jax.ShapeDtypeStruct((TOT,), jnp.float32),
    mesh=plsc.VectorSubcoreMesh(core_axis_name="c", subcore_axis_name="s"),
    scratch_types=[
        pltpu.VMEM((F * 16,), jnp.float32),
        pltpu.VMEM((CSTRIDE,), jnp.float32),
        pltpu.VMEM((CSTRIDE,), jnp.float32),
        pltpu.VMEM((CSTRIDE,), jnp.float32),
        pltpu.VMEM((CSTRIDE,), jnp.float32),
        pltpu.VMEM((CSTRIDE,), jnp.float32),
        pltpu.VMEM((CSTRIDE,), jnp.float32),
        pltpu.VMEM((CHUNK,), jnp.float32),
        pltpu.VMEM((CHUNK,), jnp.float32),
        pltpu.VMEM((CHUNK,), jnp.float32),
        pltpu.VMEM((CHUNK,), jnp.float32),
        pltpu.SemaphoreType.DMA,
        pltpu.SemaphoreType.DMA,
        pltpu.SemaphoreType.DMA,
        pltpu.SemaphoreType.DMA,
    ],
    compiler_params=pltpu.CompilerParams(needs_layout_passes=False),
)(_sc_body)


def kernel(x, unnorm_widths, unnorm_heights, unnorm_derivatives):
    bounds, coef = _prep(unnorm_widths.T, unnorm_heights.T,
                         unnorm_derivatives.T)
    y = _sc_eval(x.reshape(-1), bounds.reshape(-1), coef.reshape(-1))
    return y.reshape(BATCH, F)


# speedup vs baseline: 1.1275x; 1.0050x over previous
import functools

import jax
import jax.numpy as jnp
from jax import lax
from jax.experimental import pallas as pl
from jax.experimental.pallas import tpu as pltpu
from jax.experimental.pallas import tpu_sc as plsc

F = 512
NB = 16
BOUND = 5.0
MIN_DERIV = 0.001
MIN_BW = 0.001
MIN_BH = 0.001
BATCH = 16384

TOT = BATCH * F
NW = 32
PER_W = TOT // NW
CHUNK = 8192
NCHUNK = PER_W // CHUNK
NVEC = CHUNK // 16
CSTRIDE = F * 18
assert PER_W % CHUNK == 0 and NCHUNK % 2 == 0


def _softmax_ax0(a):
    m = jnp.max(a, axis=0, keepdims=True)
    e = jnp.exp(a - m)
    return e / jnp.sum(e, axis=0, keepdims=True)


def _prep_body(uw_ref, uh_ref, ud_ref, bounds_ref, coef_ref):
    uw = uw_ref[...]
    uh = uh_ref[...]
    ud = ud_ref[...]

    widths = _softmax_ax0(uw)
    widths = MIN_BW + (1.0 - MIN_BW * NB) * widths
    widths = widths * (2.0 * BOUND)
    heights = _softmax_ax0(uh)
    heights = MIN_BH + (1.0 - MIN_BH * NB) * heights
    heights = heights * (2.0 * BOUND)
    deriv = jnp.maximum(ud, 0.0) + jnp.log(1.0 + jnp.exp(-jnp.abs(ud))) + MIN_DERIV

    r = lax.broadcasted_iota(jnp.int32, (NB, NB), 0)
    c = lax.broadcasted_iota(jnp.int32, (NB, NB), 1)
    tri = (c <= r).astype(jnp.float32)
    cum_w = jnp.dot(tri, widths, precision=lax.Precision.HIGHEST,
                    preferred_element_type=jnp.float32)
    cum_h = jnp.dot(tri, heights, precision=lax.Precision.HIGHEST,
                    preferred_element_type=jnp.float32)

    neg_b = jnp.full((1, F), -BOUND, jnp.float32)
    knot_x = jnp.concatenate([neg_b, -BOUND + cum_w], axis=0)
    knot_y = jnp.concatenate([neg_b, -BOUND + cum_h], axis=0)

    bounds_ref[...] = jnp.concatenate(
        [knot_x[1:16], jnp.full((1, F), 1e30, jnp.float32)], axis=0)

    s = heights / widths
    d0 = deriv[:NB]
    d1 = deriv[1:]
    p = heights * (s - d0) / s
    q = heights * d0 / s
    c2 = (d0 + d1 - 2.0 * s) / s

    one = jnp.ones((1, F), jnp.float32)
    zero = jnp.zeros((1, F), jnp.float32)

    def plane(row0, inner, row17):
        return jnp.concatenate([row0, inner, row17], axis=0)

    coef = jnp.stack([
        plane(knot_x[0:1], knot_x[:NB], knot_x[16:17]),
        plane(one, 1.0 / widths, one),
        plane(zero, p, zero),
        plane(deriv[0:1], q, deriv[16:17]),
        plane(zero, c2, zero),
        plane(knot_y[0:1], knot_y[:NB], knot_y[16:17]),
    ], axis=0)
    coef_ref[...] = coef


_prep = pl.pallas_call(
    _prep_body,
    out_shape=[
        jax.ShapeDtypeStruct((16, F), jnp.float32),
        jax.ShapeDtypeStruct((6, 18, F), jnp.float32),
    ],
)


def _sc_body(x_hbm, bounds_hbm, coef_hbm, y_hbm,
             bnd_v, cf0, cf1, cf2, cf3, cf4, cf5,
             xb0, xb1, yb0, yb1, si0, si1, so0, so1):
    wid = lax.axis_index("s") * 2 + lax.axis_index("c")
    base = pl.multiple_of(wid * PER_W, CHUNK)

    pltpu.sync_copy(bounds_hbm, bnd_v)
    cfs = (cf0, cf1, cf2, cf3, cf4, cf5)
    for k in range(6):
        pltpu.sync_copy(coef_hbm.at[pl.ds(k * CSTRIDE, CSTRIDE)], cfs[k])

    xbufs = (xb0, xb1)
    ybufs = (yb0, yb1)
    isems = (si0, si1)
    osems = (so0, so1)

    def in_slice(g):
        return x_hbm.at[pl.ds(pl.multiple_of(base + g * CHUNK, CHUNK), CHUNK)]

    def out_slice(g):
        return y_hbm.at[pl.ds(pl.multiple_of(base + g * CHUNK, CHUNK), CHUNK)]

    io16 = lax.broadcasted_iota(jnp.int32, (16,), 0)

    def compute(b):
        xbuf = xbufs[b]
        ybuf = ybufs[b]

        @plsc.parallel_loop(0, NVEC, unroll=4)
        def body(i):
            off = i * 16
            fb = jnp.bitwise_and(off, F - 1)
            f = fb + io16
            xv = xbuf[pl.ds(off, 16)]
            acc = f
            for stp in (8, 4, 2, 1):
                probe = plsc.load_gather(bnd_v, [acc + (stp - 1) * F])
                acc = jnp.where(xv >= probe, acc + stp * F, acc)
            cb = acc + jnp.where(xv < -BOUND, 0, F)
            cb = cb + jnp.where(xv > BOUND, F, 0)
            xk = plsc.load_gather(cf0, [cb])
            iw = plsc.load_gather(cf1, [cb])
            pp = plsc.load_gather(cf2, [cb])
            qq = plsc.load_gather(cf3, [cb])
            cc = plsc.load_gather(cf4, [cb])
            yk = plsc.load_gather(cf5, [cb])
            xi = (xv - xk) * iw
            t = xi * (1.0 - xi)
            num = xi * (pp * xi + qq)
            den = 1.0 + cc * t
            ybuf[pl.ds(off, 16)] = yk + num / den

    pltpu.async_copy(in_slice(0), xb0, si0)
    pltpu.async_copy(in_slice(1), xb1, si1)

    def outer(k, carry):
        g2 = k * 2
        for b in (0, 1):
            g = g2 + b
            pltpu.make_async_copy(in_slice(0), xbufs[b], isems[b]).wait()

            @pl.when(g2 >= 2)
            def _wait_out():
                pltpu.make_async_copy(ybufs[b], out_slice(0), osems[b]).wait()

            compute(b)
            pltpu.async_copy(ybufs[b], out_slice(g), osems[b])

            @pl.when(g2 + 2 < NCHUNK)
            def _next_in():
                pltpu.async_copy(in_slice(g + 2), xbufs[b], isems[b])
        return carry

    lax.fori_loop(0, NCHUNK // 2, outer, 0)

    pltpu.make_async_copy(yb0, out_slice(0), so0).wait()
    pltpu.make_async_copy(yb1, out_slice(0), so1).wait()


_sc_eval = functools.partial(
    pl.kernel,
    out_type=jax.ShapeDtypeStruct((TOT,), jnp.float32),
    mesh=plsc.VectorSubcoreMesh(core_axis_name="c", subcore_axis_name="s"),
    scratch_types=[
        pltpu.VMEM((F * 16,), jnp.float32),
        pltpu.VMEM((CSTRIDE,), jnp.float32),
        pltpu.VMEM((CSTRIDE,), jnp.float32),
        pltpu.VMEM((CSTRIDE,), jnp.float32),
        pltpu.VMEM((CSTRIDE,), jnp.float32),
        pltpu.VMEM((CSTRIDE,), jnp.float32),
        pltpu.VMEM((CSTRIDE,), jnp.float32),
        pltpu.VMEM((CHUNK,), jnp.float32),
        pltpu.VMEM((CHUNK,), jnp.float32),
        pltpu.VMEM((CHUNK,), jnp.float32),
        pltpu.VMEM((CHUNK,), jnp.float32),
        pltpu.SemaphoreType.DMA,
        pltpu.SemaphoreType.DMA,
        pltpu.SemaphoreType.DMA,
        pltpu.SemaphoreType.DMA,
    ],
    compiler_params=pltpu.CompilerParams(needs_layout_passes=False),
)(_sc_body)


def kernel(x, unnorm_widths, unnorm_heights, unnorm_derivatives):
    bounds, coef = _prep(unnorm_widths.T, unnorm_heights.T,
                         unnorm_derivatives.T)
    y = _sc_eval(x.reshape(-1), bounds.reshape(-1), coef.reshape(-1))
    return y.reshape(BATCH, F)
